# baseline (speedup 1.0000x reference)
import jax
import jax.numpy as jnp
from jax.experimental import pallas as pl
from jax.experimental.pallas import tpu as pltpu

_BM = 64


def _mv_block(a_ref, x_ref, o_ref):
    i = pl.program_id(0)
    r = jax.lax.dot_general(
        a_ref[...], x_ref[...],
        dimension_numbers=(((1,), (1,)), ((), ())),
        preferred_element_type=jnp.float32,
    )
    o_ref[pl.ds(i, 1), :] = r.reshape(1, _BM)


def kernel(A, x):
    m, k = A.shape
    x2 = x.reshape(1, k)
    out = pl.pallas_call(
        _mv_block,
        grid=(m // _BM,),
        in_specs=[
            pl.BlockSpec((_BM, k), lambda i: (i, 0)),
            pl.BlockSpec((1, k), lambda i: (0, 0)),
        ],
        out_specs=pl.BlockSpec((m // _BM, _BM), lambda i: (0, 0)),
        out_shape=jax.ShapeDtypeStruct((m // _BM, _BM), jnp.float32),
    )(A, x2)
    return out.reshape(m)


# speedup vs baseline: 1.0786x; 1.0786x over previous
import functools

import jax
import jax.numpy as jnp
from jax import lax
from jax.experimental import pallas as pl
from jax.experimental.pallas import tpu as pltpu
from jax.experimental.pallas import tpu_sc as plsc

_M = 16384
_K = 16384
_SC_ROWS = 1024
_NW = 32
_RPW = _SC_ROWS // _NW
_CR = 4
_BM = 128


def _mv_block(a_ref, x_ref, o_ref):
    i = pl.program_id(0)
    r = jax.lax.dot_general(
        a_ref[...], x_ref[...],
        dimension_numbers=(((1,), (1,)), ((), ())),
        preferred_element_type=jnp.float32,
    )
    o_ref[pl.ds(i, 1), :] = r.reshape(1, _BM)


def _tc_part(A, x2):
    m_tc = _M - _SC_ROWS
    off = _SC_ROWS // _BM
    nb = m_tc // _BM
    out = pl.pallas_call(
        _mv_block,
        grid=(nb,),
        in_specs=[
            pl.BlockSpec((_BM, _K), lambda i: (i + off, 0)),
            pl.BlockSpec((1, _K), lambda i: (0, 0)),
        ],
        out_specs=pl.BlockSpec((nb, _BM), lambda i: (0, 0)),
        out_shape=jax.ShapeDtypeStruct((nb, _BM), jnp.float32),
    )(A, x2)
    return out.reshape(m_tc)


_sc_mesh = plsc.VectorSubcoreMesh(core_axis_name="c", subcore_axis_name="s")


@functools.partial(
    pl.kernel,
    mesh=_sc_mesh,
    out_type=jax.ShapeDtypeStruct((_SC_ROWS,), jnp.float32),
    scratch_types=[
        pltpu.VMEM((_K,), jnp.float32),
        pltpu.VMEM((_CR, _K), jnp.float32),
        pltpu.VMEM((_RPW,), jnp.float32),
    ],
)
def _sc_mv(a_hbm, x_hbm, o_hbm, x_v, a_v, out_v):
    wid = lax.axis_index("s") * 2 + lax.axis_index("c")
    base = wid * _RPW
    pltpu.sync_copy(x_hbm, x_v)
    lane = lax.iota(jnp.int32, 16)

    for g in range(_RPW // 16):
        out16 = jnp.zeros((16,), jnp.float32)
        for cc in range(16 // _CR):
            pltpu.sync_copy(
                a_hbm.at[pl.ds(base + g * 16 + cc * _CR, _CR)], a_v
            )

            def inner(c, accs):
                xc = x_v[pl.ds(c * 16, 16)]
                return tuple(
                    accs[j] + a_v[j, pl.ds(c * 16, 16)] * xc
                    for j in range(_CR)
                )

            zeros = tuple(jnp.zeros((16,), jnp.float32) for _ in range(_CR))
            accs = lax.fori_loop(0, _K // 16, inner, zeros)
            for j in range(_CR):
                v = accs[j]
                s = v[0]
                for t in range(1, 16):
                    s = s + v[t]
                out16 = jnp.where(
                    lane == cc * _CR + j,
                    jnp.zeros((16,), jnp.float32) + s,
                    out16,
                )
        out_v[pl.ds(g * 16, 16)] = out16
    pltpu.sync_copy(out_v, o_hbm.at[pl.ds(base, _RPW)])


def kernel(A, x):
    x2 = x.reshape(1, _K)
    sc_out = _sc_mv(A, x)
    tc_out = _tc_part(A, x2)
    return jnp.concatenate([sc_out, tc_out])


# speedup vs baseline: 1.1516x; 1.0676x over previous
import jax
import jax.numpy as jnp
from jax.experimental import pallas as pl

_BM = 128


def _mv_block(a_ref, x_ref, o_ref):
    i = pl.program_id(0)
    r = jax.lax.dot_general(
        a_ref[...], x_ref[...],
        dimension_numbers=(((1,), (1,)), ((), ())),
        preferred_element_type=jnp.float32,
    )
    o_ref[pl.ds(i, 1), :] = r.reshape(1, _BM)


def kernel(A, x):
    m, k = A.shape
    x2 = x.reshape(1, k)
    nb = m // _BM
    out = pl.pallas_call(
        _mv_block,
        grid=(nb,),
        in_specs=[
            pl.BlockSpec((_BM, k), lambda i: (i, 0)),
            pl.BlockSpec((1, k), lambda i: (0, 0)),
        ],
        out_specs=pl.BlockSpec((nb, _BM), lambda i: (0, 0)),
        out_shape=jax.ShapeDtypeStruct((nb, _BM), jnp.float32),
    )(A, x2)
    return out.reshape(m)
